# baseline (speedup 1.0000x reference)
import jax
import jax.numpy as jnp
from jax.experimental import pallas as pl

N_CPT = 128
N_ITEM = 20000
N_USER = 100000


def _gat(h, src, dst, W, a, num_nodes):
    z = h @ W.T
    zs = jnp.take(z, src, axis=0)
    zd = jnp.take(z, dst, axis=0)
    e = (jnp.concatenate([zs, zd], axis=1) @ a.T)[:, 0]
    m = jax.ops.segment_max(e, dst, num_segments=num_nodes)
    m = jnp.where(jnp.isfinite(m), m, 0.0)
    ex = jnp.exp(e - jnp.take(m, dst))
    den = jax.ops.segment_sum(ex, dst, num_segments=num_nodes)
    alpha = ex / jnp.take(den, dst)
    return jax.ops.segment_sum(alpha[:, None] * zs, dst, num_segments=num_nodes)


def _copy_k(x_ref, o_ref):
    o_ref[...] = x_ref[...]


def kernel(kn_emb, exer_emb, all_stu_emb, W_dir, a_dir, W_und, a_und, W_ke, a_ke, W_ek, a_ek, W_ue, a_ue, W_eu, a_eu, k_w1, k_b1, k_w2, k_b2, k_w3, k_b3, e_w1, e_b1, e_w2, e_b2, edges_dir, edges_undir, edges_ke, edges_eu):
    kn_emb = pl.pallas_call(
        _copy_k, out_shape=jax.ShapeDtypeStruct(kn_emb.shape, kn_emb.dtype)
    )(kn_emb)
    k_directed = _gat(kn_emb, edges_dir[0], edges_dir[1], W_dir, a_dir, N_CPT)
    k_undirected = _gat(kn_emb, edges_undir[0], edges_undir[1], W_und, a_und, N_CPT)
    e_k = jnp.concatenate([exer_emb, kn_emb], axis=0)
    n_ek = N_ITEM + N_CPT
    k_from_e = _gat(e_k, edges_ke[0], edges_ke[1], W_ke, a_ke, n_ek)
    e_from_k = _gat(e_k, edges_ke[1], edges_ke[0], W_ek, a_ek, n_ek)
    e_u = jnp.concatenate([exer_emb, all_stu_emb], axis=0)
    n_eu = N_ITEM + N_USER
    u_from_e = _gat(e_u, edges_eu[0], edges_eu[1], W_ue, a_ue, n_eu)
    e_from_u = _gat(e_u, edges_eu[1], edges_eu[0], W_eu, a_eu, n_eu)
    A = kn_emb; B = k_directed; C = k_undirected; Dk = k_from_e[N_ITEM:]
    s1 = jnp.concatenate([A, B], axis=1) @ k_w1.T + k_b1
    s2 = jnp.concatenate([A, C], axis=1) @ k_w2.T + k_b2
    s3 = jnp.concatenate([A, Dk], axis=1) @ k_w3.T + k_b3
    score = jax.nn.softmax(jnp.concatenate([s1, s2, s3], axis=1), axis=1)
    kn_out = A + score[:, 0:1] * B + score[:, 1:2] * C + score[:, 2:3] * Dk
    Ae = exer_emb; Be = e_from_k[:N_ITEM]; Ce = e_from_u[:N_ITEM]
    t1 = jnp.concatenate([Ae, Be], axis=1) @ e_w1.T + e_b1
    t2 = jnp.concatenate([Ae, Ce], axis=1) @ e_w2.T + e_b2
    sc = jax.nn.softmax(jnp.concatenate([t1, t2], axis=1), axis=1)
    exer_out = Ae + sc[:, 0:1] * Be + sc[:, 1:2] * Ce
    stu_out = all_stu_emb + u_from_e[N_ITEM:]
    return (kn_out, exer_out, stu_out)


# speedup vs baseline: 7.3447x; 7.3447x over previous
import functools

import jax
import jax.numpy as jnp
from jax import lax
from jax.experimental import pallas as pl
from jax.experimental.pallas import tpu as pltpu
from jax.experimental.pallas import tpu_sc as plsc

N_CPT = 128
N_ITEM = 20000
N_USER = 100000
D = 128
DP = 256
K = 512
BLKA = 512

NC, NS = 2, 16
NW = NC * NS


def _ceil_to(x, m):
    return ((x + m - 1) // m) * m


def _prep_body(h_ref, wt_ref, a1_ref, o_ref, *, n_valid):
    b = pl.program_id(0)
    z = jnp.dot(h_ref[...], wt_ref[...], preferred_element_type=jnp.float32)
    s = jnp.dot(z, a1_ref[...], preferred_element_type=jnp.float32)
    rid = b * BLKA + lax.broadcasted_iota(jnp.int32, (BLKA, 1), 0)
    w = jnp.exp(s) * (rid < n_valid).astype(jnp.float32)
    o_ref[...] = jnp.concatenate(
        [z * w, w, jnp.zeros((BLKA, DP - D - 1), jnp.float32)], axis=1)


def _prep(h_pad, W, a, n_valid):
    n_pad = h_pad.shape[0]
    return pl.pallas_call(
        functools.partial(_prep_body, n_valid=n_valid),
        grid=(n_pad // BLKA,),
        in_specs=[
            pl.BlockSpec((BLKA, D), lambda b: (b, 0)),
            pl.BlockSpec((D, D), lambda b: (0, 0)),
            pl.BlockSpec((D, 1), lambda b: (0, 0)),
        ],
        out_specs=pl.BlockSpec((BLKA, DP), lambda b: (b, 0)),
        out_shape=jax.ShapeDtypeStruct((n_pad, DP), jnp.float32),
    )(h_pad, W.T, a[:, :D].T)


def _sc_gather(table, idx, ch):
    e_pad = idx.shape[0]
    b_per_w = e_pad // NW
    n_iter = b_per_w // ch
    mesh = plsc.VectorSubcoreMesh(core_axis_name="c", subcore_axis_name="s")

    @functools.partial(
        pl.kernel, mesh=mesh,
        out_type=jax.ShapeDtypeStruct((e_pad, DP), jnp.float32),
        scratch_types=[
            pltpu.VMEM((ch,), jnp.int32),
            pltpu.VMEM((ch, DP), jnp.float32),
            pltpu.SemaphoreType.DMA,
        ],
    )
    def k(table_hbm, idx_hbm, out_hbm, idx_v, rows_v, sem):
        wid = lax.axis_index("s") * NC + lax.axis_index("c")
        base = wid * b_per_w

        def step(j, carry):
            off = base + j * ch
            pltpu.sync_copy(idx_hbm.at[pl.ds(off, ch)], idx_v)
            pltpu.async_copy(table_hbm.at[idx_v], rows_v, sem).wait()
            pltpu.sync_copy(rows_v, out_hbm.at[pl.ds(off, ch)])
            return carry

        lax.fori_loop(0, n_iter, step, 0)

    return k(table, idx)


def _seg_body(lo_ref, cnt_ref, dst_ref, vals_ref, o_ref, dscr, vscr, s1, s2):
    b = pl.program_id(0)
    lo = lo_ref[b]
    cnt = cnt_ref[b]
    o_ref[...] = jnp.zeros((128, DP), jnp.float32)

    def step(i, carry):
        c = lo + i
        cp_d = pltpu.make_async_copy(dst_ref.at[pl.ds(c, 1)], dscr, s1)
        cp_v = pltpu.make_async_copy(vals_ref.at[pl.ds(c * K, K)], vscr, s2)
        cp_d.start()
        cp_v.start()
        cp_d.wait()
        cp_v.wait()
        row_ids = lax.broadcasted_iota(jnp.int32, (128, K), 0) + b * 128
        oh = (row_ids == dscr[...]).astype(jnp.float32)
        o_ref[...] += jnp.dot(oh, vscr[...], preferred_element_type=jnp.float32)
        return carry

    lax.fori_loop(0, cnt, step, 0)


def _seg_reduce(c_lo, c_cnt, dst2d, vals, n_blocks):
    return pl.pallas_call(
        _seg_body,
        grid=(n_blocks,),
        in_specs=[
            pl.BlockSpec(memory_space=pltpu.SMEM),
            pl.BlockSpec(memory_space=pltpu.SMEM),
            pl.BlockSpec(memory_space=pl.ANY),
            pl.BlockSpec(memory_space=pl.ANY),
        ],
        out_specs=pl.BlockSpec((128, DP), lambda b: (b, 0)),
        out_shape=jax.ShapeDtypeStruct((n_blocks * 128, DP), jnp.float32),
        scratch_shapes=[
            pltpu.VMEM((1, K), jnp.int32),
            pltpu.VMEM((K, DP), jnp.float32),
            pltpu.SemaphoreType.DMA,
            pltpu.SemaphoreType.DMA,
        ],
    )(c_lo, c_cnt, dst2d, vals)


def _gat(h_pad, n_valid, W, a, src, dst, n_dst, ch):
    e = src.shape[0]
    e_pad = _ceil_to(e, max(NW * ch, K))
    perm = jnp.argsort(dst)
    src_s = jnp.take(src, perm).astype(jnp.int32)
    dst_s = jnp.take(dst, perm).astype(jnp.int32)
    pad = e_pad - e
    src_p = jnp.concatenate([src_s, jnp.full((pad,), n_valid, jnp.int32)])
    dst_p = jnp.concatenate([dst_s, jnp.full((pad,), n_dst - 1, jnp.int32)])
    n_blocks = _ceil_to(n_dst, 128) // 128
    rps = jnp.searchsorted(dst_p, jnp.arange(n_blocks + 1) * 128).astype(jnp.int32)
    rp0, rp1 = rps[:-1], rps[1:]
    c_lo = rp0 // K
    c_cnt = jnp.where(rp1 > rp0, (rp1 - 1) // K - rp0 // K + 1, 0).astype(jnp.int32)

    table = _prep(h_pad, W, a, n_valid)
    vals = _sc_gather(table, src_p, ch)
    return _seg_reduce(c_lo, c_cnt, dst_p.reshape(e_pad // K, K), vals, n_blocks)


def _fin(acc):
    den = acc[:, D:D + 1]
    safe = jnp.where(den > 0, den, 1.0)
    return jnp.where(den > 0, acc[:, :D] / safe, 0.0)


def _kfuse_body(a_ref, b_ref, c_ref, d_ref, w1a, w1b, w2a, w2b, w3a, w3b,
                b1, b2, b3, o_ref):
    A = a_ref[...]
    B = _fin(b_ref[...])
    C = _fin(c_ref[...])
    Dk = _fin(d_ref[...])
    s1 = jnp.dot(A, w1a[...]) + jnp.dot(B, w1b[...]) + b1[...]
    s2 = jnp.dot(A, w2a[...]) + jnp.dot(C, w2b[...]) + b2[...]
    s3 = jnp.dot(A, w3a[...]) + jnp.dot(Dk, w3b[...]) + b3[...]
    m = jnp.maximum(jnp.maximum(s1, s2), s3)
    e1, e2, e3 = jnp.exp(s1 - m), jnp.exp(s2 - m), jnp.exp(s3 - m)
    den = e1 + e2 + e3
    o_ref[...] = A + (e1 / den) * B + (e2 / den) * C + (e3 / den) * Dk


def _efuse_body(a_ref, b_ref, c_ref, w1a, w1b, w2a, w2b, b1, b2, o_ref):
    A = a_ref[...]
    B = _fin(b_ref[...])
    C = _fin(c_ref[...])
    t1 = jnp.dot(A, w1a[...]) + jnp.dot(B, w1b[...]) + b1[...]
    t2 = jnp.dot(A, w2a[...]) + jnp.dot(C, w2b[...]) + b2[...]
    m = jnp.maximum(t1, t2)
    e1, e2 = jnp.exp(t1 - m), jnp.exp(t2 - m)
    den = e1 + e2
    o_ref[...] = A + (e1 / den) * B + (e2 / den) * C


def _sfuse_body(a_ref, b_ref, o_ref):
    o_ref[...] = a_ref[...] + _fin(b_ref[...])


def kernel(kn_emb, exer_emb, all_stu_emb, W_dir, a_dir, W_und, a_und, W_ke,
           a_ke, W_ek, a_ek, W_ue, a_ue, W_eu, a_eu, k_w1, k_b1, k_w2, k_b2,
           k_w3, k_b3, e_w1, e_b1, e_w2, e_b2, edges_dir, edges_undir,
           edges_ke, edges_eu):
    f32 = jnp.float32

    kn_pad = jnp.pad(kn_emb, ((0, _ceil_to(N_CPT + 1, BLKA) - N_CPT), (0, 0)))
    e_k = jnp.concatenate([exer_emb, kn_emb], axis=0)
    n_ek = N_ITEM + N_CPT
    ek_pad = jnp.pad(e_k, ((0, _ceil_to(n_ek + 1, BLKA) - n_ek), (0, 0)))
    e_u = jnp.concatenate([exer_emb, all_stu_emb], axis=0)
    n_eu = N_ITEM + N_USER
    eu_pad = jnp.pad(e_u, ((0, _ceil_to(n_eu + 1, BLKA) - n_eu), (0, 0)))

    acc_dir = _gat(kn_pad, N_CPT, W_dir, a_dir,
                   edges_dir[0], edges_dir[1], N_CPT, 64)
    acc_und = _gat(kn_pad, N_CPT, W_und, a_und,
                   edges_undir[0], edges_undir[1], N_CPT, 64)
    acc_kfe = _gat(ek_pad, n_ek, W_ke, a_ke,
                   edges_ke[0], edges_ke[1] - N_ITEM, N_CPT, 128)
    acc_efk = _gat(ek_pad, n_ek, W_ek, a_ek,
                   edges_ke[1], edges_ke[0], N_ITEM, 128)
    acc_ufe = _gat(eu_pad, n_eu, W_ue, a_ue,
                   edges_eu[0], edges_eu[1] - N_ITEM, N_USER, 128)
    acc_efu = _gat(eu_pad, n_eu, W_eu, a_eu,
                   edges_eu[1], edges_eu[0], N_ITEM, 128)

    kn_out = pl.pallas_call(
        _kfuse_body,
        out_shape=jax.ShapeDtypeStruct((N_CPT, D), f32),
    )(kn_emb, acc_dir, acc_und, acc_kfe,
      k_w1[:, :D].T, k_w1[:, D:].T, k_w2[:, :D].T, k_w2[:, D:].T,
      k_w3[:, :D].T, k_w3[:, D:].T,
      k_b1.reshape(1, 1), k_b2.reshape(1, 1), k_b3.reshape(1, 1))

    eblk = 2000
    exer_out = pl.pallas_call(
        _efuse_body,
        grid=(N_ITEM // eblk,),
        in_specs=[
            pl.BlockSpec((eblk, D), lambda b: (b, 0)),
            pl.BlockSpec((eblk, DP), lambda b: (b, 0)),
            pl.BlockSpec((eblk, DP), lambda b: (b, 0)),
            pl.BlockSpec((D, 1), lambda b: (0, 0)),
            pl.BlockSpec((D, 1), lambda b: (0, 0)),
            pl.BlockSpec((D, 1), lambda b: (0, 0)),
            pl.BlockSpec((D, 1), lambda b: (0, 0)),
            pl.BlockSpec((1, 1), lambda b: (0, 0)),
            pl.BlockSpec((1, 1), lambda b: (0, 0)),
        ],
        out_specs=pl.BlockSpec((eblk, D), lambda b: (b, 0)),
        out_shape=jax.ShapeDtypeStruct((N_ITEM, D), f32),
    )(exer_emb, acc_efk, acc_efu,
      e_w1[:, :D].T, e_w1[:, D:].T, e_w2[:, :D].T, e_w2[:, D:].T,
      e_b1.reshape(1, 1), e_b2.reshape(1, 1))

    sblk = 2000
    stu_out = pl.pallas_call(
        _sfuse_body,
        grid=(N_USER // sblk,),
        in_specs=[
            pl.BlockSpec((sblk, D), lambda b: (b, 0)),
            pl.BlockSpec((sblk, DP), lambda b: (b, 0)),
        ],
        out_specs=pl.BlockSpec((sblk, D), lambda b: (b, 0)),
        out_shape=jax.ShapeDtypeStruct((N_USER, D), f32),
    )(all_stu_emb, acc_ufe)

    return (kn_out, exer_out, stu_out)
